# Initial kernel scaffold; baseline (speedup 1.0000x reference)
import jax
import jax.numpy as jnp
from jax.experimental import pallas as pl


def kernel(atomic_ns, edge_index, coords, batch_node_vec, node_emb_table):
    raise NotImplementedError("write your pallas kernel here")


# speedup vs baseline: 7.6262x; 7.6262x over previous
import functools

import jax
import jax.numpy as jnp
from jax import lax
from jax.experimental import pallas as pl
from jax.experimental.pallas import tpu as pltpu
from jax.experimental.pallas import tpu_sc as plsc

N_NODES = 100000
N_EDGES = 1600000
POSS_ELEMS = 100
H_NF = 128
CUTOFF = 10.0
N_GAUSSIANS = 50
ORIGIN = 0.0

_STEP = (CUTOFF - ORIGIN) / (N_GAUSSIANS - 1)
_COEFF = -0.5 / _STEP**2


_NC = 2
_NS = 16
_NW = _NC * _NS
_EPW = N_EDGES // _NW
_CHUNK = 10000
_NCHUNK = _EPW // _CHUNK


def _sc_d2_body(src_hbm, dst_hbm, x_hbm, y_hbm, z_hbm,
                d2x_hbm, d2y_hbm, d2z_hbm, tab_v, si_v, di_v, o_v):
    wid = lax.axis_index("s") * _NC + lax.axis_index("c")
    base = wid * _EPW
    tabs = (x_hbm, y_hbm, z_hbm)
    outs = (d2x_hbm, d2y_hbm, d2z_hbm)
    for comp in range(3):
        pltpu.sync_copy(tabs[comp], tab_v)
        for k in range(_NCHUNK):
            off = base + k * _CHUNK
            pltpu.sync_copy(src_hbm.at[pl.ds(off, _CHUNK)], si_v)
            pltpu.sync_copy(dst_hbm.at[pl.ds(off, _CHUNK)], di_v)

            @plsc.parallel_loop(0, _CHUNK // 16, 1, unroll=8)
            def _gather_step(i):
                sl = pl.ds(i * 16, 16)
                a = plsc.load_gather(tab_v, [si_v[sl]])
                b = plsc.load_gather(tab_v, [di_v[sl]])
                d = a - b
                o_v[sl] = d * d

            pltpu.sync_copy(o_v, outs[comp].at[pl.ds(off, _CHUNK)])


@jax.jit
def _sc_d2(src, dst, x, y, z):
    edge_f32 = jax.ShapeDtypeStruct((N_EDGES,), jnp.float32)
    fn = pl.kernel(
        _sc_d2_body,
        out_type=(edge_f32, edge_f32, edge_f32),
        mesh=plsc.VectorSubcoreMesh(core_axis_name="c", subcore_axis_name="s"),
        compiler_params=pltpu.CompilerParams(needs_layout_passes=False),
        scratch_types=[
            pltpu.VMEM((N_NODES,), jnp.float32),
            pltpu.VMEM((_CHUNK,), jnp.int32),
            pltpu.VMEM((_CHUNK,), jnp.int32),
            pltpu.VMEM((_CHUNK,), jnp.float32),
        ],
    )
    return fn(src, dst, x, y, z)


_NB = 4096


def _node_body(ids_ref, tab_ref, out_ref):
    ids = ids_ref[...]
    onehot = (ids[:, None]
              == lax.broadcasted_iota(jnp.int32, (1, 128), 1)).astype(jnp.float32)
    out_ref[...] = jnp.dot(onehot, tab_ref[...],
                           preferred_element_type=jnp.float32)


@jax.jit
def _node_embs_call(atomic_ns, table128):
    grid = (N_NODES + _NB - 1) // _NB
    return pl.pallas_call(
        _node_body,
        grid=(grid,),
        in_specs=[
            pl.BlockSpec((_NB,), lambda i: (i,)),
            pl.BlockSpec((128, H_NF), lambda i: (0, 0)),
        ],
        out_specs=pl.BlockSpec((_NB, H_NF), lambda i: (i, 0)),
        out_shape=jax.ShapeDtypeStruct((N_NODES, H_NF), jnp.float32),
    )(atomic_ns, table128)


_EB = 16384


def _smear_body(d2x_ref, d2y_ref, d2z_ref, w_ref, e_ref):
    s = d2x_ref[...] + d2y_ref[...] + d2z_ref[...] + 1e-12
    w = jnp.sqrt(s)
    w_ref[...] = w
    offs = lax.broadcasted_iota(jnp.int32, (1, N_GAUSSIANS), 1).astype(
        jnp.float32) * _STEP
    d = w[:, None] - offs
    e_ref[...] = jnp.exp(_COEFF * (d * d))


@jax.jit
def _smear_call(d2x, d2y, d2z):
    grid = ((N_EDGES + _EB - 1) // _EB,)
    return pl.pallas_call(
        _smear_body,
        grid=grid,
        in_specs=[pl.BlockSpec((_EB,), lambda i: (i,))] * 3,
        out_specs=[
            pl.BlockSpec((_EB,), lambda i: (i,)),
            pl.BlockSpec((_EB, N_GAUSSIANS), lambda i: (i, 0)),
        ],
        out_shape=[
            jax.ShapeDtypeStruct((N_EDGES,), jnp.float32),
            jax.ShapeDtypeStruct((N_EDGES, N_GAUSSIANS), jnp.float32),
        ],
    )(d2x, d2y, d2z)


def kernel(atomic_ns, edge_index, coords, batch_node_vec, node_emb_table):
    src = edge_index[0]
    dst = edge_index[1]
    xyz = coords.T
    d2x, d2y, d2z = _sc_d2(src, dst, xyz[0], xyz[1], xyz[2])
    table128 = jnp.pad(node_emb_table, ((0, 128 - POSS_ELEMS), (0, 0)))
    node_embs = _node_embs_call(atomic_ns, table128)
    edge_weights, edge_embs = _smear_call(d2x, d2y, d2z)
    return (node_embs, edge_embs, edge_weights)
